# baseline (speedup 1.0000x reference)
import functools

import jax
import jax.numpy as jnp
from jax import lax
from jax.experimental import pallas as pl
from jax.experimental.pallas import tpu as pltpu
from jax.experimental.pallas import tpu_sc as plsc

N = 100000
D = 128
NDOM = 16
NC = 2
NS = 16
L = 16
NW = NC * NS
C = 160
NCHUNK = N // C
NBUF = 3
SLOTS = 21
ROUNDS = SLOTS // NBUF
PF = 2
NCG = D // L

_mesh = plsc.VectorSubcoreMesh(core_axis_name="c", subcore_axis_name="s")


@functools.partial(
    pl.kernel,
    out_type=jax.ShapeDtypeStruct((N, D), jnp.float32),
    mesh=_mesh,
    compiler_params=pltpu.CompilerParams(needs_layout_passes=False),
    scratch_types=(
        [pltpu.VMEM((NBUF, C, D), jnp.float32)]
        + [pltpu.VMEM((NBUF, C, D), jnp.float32)]
        + [pltpu.VMEM((C,), jnp.int32)] * NBUF
        + [pltpu.VMEM((NDOM * D,), jnp.float32)]
        + [pltpu.SemaphoreType.DMA] * NBUF
        + [pltpu.SemaphoreType.DMA] * NBUF
    ),
)
def _align_prompt(emb_hbm, names_hbm, prompt_hbm, out_hbm,
                  emb_v, out_v, *rest):
    names_v = rest[:NBUF]
    prompt_v = rest[NBUF]
    in_sems = rest[NBUF + 1:NBUF + 1 + NBUF]
    out_sems = rest[NBUF + 1 + NBUF:]
    wid = lax.axis_index("s") * NC + lax.axis_index("c")

    iota = lax.iota(jnp.int32, L)
    cols = [iota + (c * L) for c in range(NCG)]

    def in_copy(b, k):
        base = k * C
        return (
            pltpu.make_async_copy(names_hbm.at[pl.ds(base, C)], names_v[b],
                                  in_sems[b]),
            pltpu.make_async_copy(emb_hbm.at[pl.ds(base, C), :], emb_v.at[b],
                                  in_sems[b]),
        )

    def out_copy(b, k):
        base = k * C
        return pltpu.make_async_copy(out_v.at[b],
                                     out_hbm.at[pl.ds(base, C), :],
                                     out_sems[b])

    def compute(b):
        nref = names_v[b]

        @plsc.parallel_loop(0, C // L)
        def _grp(g):
            base16 = nref[pl.ds(g * L, L)] * D
            for r in range(L):
                row = g * L + r
                bvec = jnp.take_along_axis(
                    base16, jnp.full((L,), r, jnp.int32), axis=0)
                for c in range(NCG):
                    sel = plsc.load_gather(prompt_v, [bvec + cols[c]])
                    x = emb_v[b, row, pl.ds(c * L, L)]
                    out_v[b, row, pl.ds(c * L, L)] = x * sel

    for s in range(PF):
        for cp in in_copy(s, wid + NW * s):
            cp.start()
    pltpu.sync_copy(prompt_hbm, prompt_v)

    def ring_round(j, _):
        for b in range(NBUF):
            s = NBUF * j + b
            k = wid + NW * s

            @pl.when(k + PF * NW < NCHUNK)
            def _(b=b, k=k):
                for cp in in_copy((b + PF) % NBUF, k + PF * NW):
                    cp.start()

            @pl.when(s >= NBUF)
            def _(b=b, k=k):
                out_copy(b, k - NBUF * NW).wait()

            @pl.when(k < NCHUNK)
            def _(b=b, k=k):
                for cp in in_copy(b, k):
                    cp.wait()
                compute(b)
                out_copy(b, k).start()
        return 0

    lax.fori_loop(0, ROUNDS, ring_round, 0, unroll=False)

    for s in range(SLOTS - NBUF, SLOTS):
        k = wid + NW * s

        @pl.when(k < NCHUNK)
        def _(s=s, k=k):
            out_copy(s % NBUF, k).wait()


def kernel(graph_embedding, names, prompt):
    names_i = names.astype(jnp.int32)
    prompt_flat = prompt.reshape(-1)
    return _align_prompt(graph_embedding, names_i, prompt_flat)


# speedup vs baseline: 2.1847x; 2.1847x over previous
import functools

import jax
import jax.numpy as jnp
from jax import lax
from jax.experimental import pallas as pl
from jax.experimental.pallas import tpu as pltpu
from jax.experimental.pallas import tpu_sc as plsc

N = 100000
D = 128
NDOM = 16
NC = 2
NS = 16
L = 16
NW = NC * NS
C = 160
NCHUNK = N // C
NBUF = 3
SLOTS = 21
ROUNDS = SLOTS // NBUF
PF = 2
NCG = D // L

_mesh = plsc.VectorSubcoreMesh(core_axis_name="c", subcore_axis_name="s")


@functools.partial(
    pl.kernel,
    out_type=jax.ShapeDtypeStruct((N, D), jnp.float32),
    mesh=_mesh,
    compiler_params=pltpu.CompilerParams(needs_layout_passes=False),
    scratch_types=(
        [pltpu.VMEM((NBUF, C, D), jnp.float32)]
        + [pltpu.VMEM((NBUF, C, D), jnp.float32)]
        + [pltpu.VMEM((C,), jnp.int32)] * NBUF
        + [pltpu.VMEM((NDOM * D,), jnp.float32)]
        + [pltpu.SemaphoreType.DMA] * NBUF
        + [pltpu.SemaphoreType.DMA] * NBUF
    ),
)
def _align_prompt(emb_hbm, names_hbm, prompt_hbm, out_hbm,
                  emb_v, out_v, *rest):
    names_v = rest[:NBUF]
    prompt_v = rest[NBUF]
    in_sems = rest[NBUF + 1:NBUF + 1 + NBUF]
    out_sems = rest[NBUF + 1 + NBUF:]
    wid = lax.axis_index("s") * NC + lax.axis_index("c")

    iota = lax.iota(jnp.int32, L)
    cols = [iota + (c * L) for c in range(NCG)]

    def in_copy(b, k):
        base = k * C
        return (
            pltpu.make_async_copy(names_hbm.at[pl.ds(base, C)], names_v[b],
                                  in_sems[b]),
            pltpu.make_async_copy(emb_hbm.at[pl.ds(base, C), :], emb_v.at[b],
                                  in_sems[b]),
        )

    def out_copy(b, k):
        base = k * C
        return pltpu.make_async_copy(out_v.at[b],
                                     out_hbm.at[pl.ds(base, C), :],
                                     out_sems[b])

    def compute(b):
        nref = names_v[b]

        @plsc.parallel_loop(0, C, unroll=2)
        def _row(r):
            n16 = plsc.load_gather(nref, [jnp.full((L,), r, jnp.int32)])
            bvec = n16 * D
            for c in range(NCG):
                sel = plsc.load_gather(prompt_v, [bvec + cols[c]])
                x = emb_v[b, r, pl.ds(c * L, L)]
                out_v[b, r, pl.ds(c * L, L)] = x * sel

    for s in range(PF):
        for cp in in_copy(s, wid + NW * s):
            cp.start()
    pltpu.sync_copy(prompt_hbm, prompt_v)

    def ring_round(j, _):
        for b in range(NBUF):
            s = NBUF * j + b
            k = wid + NW * s

            @pl.when(k + PF * NW < NCHUNK)
            def _(b=b, k=k):
                for cp in in_copy((b + PF) % NBUF, k + PF * NW):
                    cp.start()

            @pl.when(s >= NBUF)
            def _(b=b, k=k):
                out_copy(b, k - NBUF * NW).wait()

            @pl.when(k < NCHUNK)
            def _(b=b, k=k):
                for cp in in_copy(b, k):
                    cp.wait()
                compute(b)
                out_copy(b, k).start()
        return 0

    lax.fori_loop(0, ROUNDS, ring_round, 0, unroll=False)

    for s in range(SLOTS - NBUF, SLOTS):
        k = wid + NW * s

        @pl.when(k < NCHUNK)
        def _(s=s, k=k):
            out_copy(s % NBUF, k).wait()


def kernel(graph_embedding, names, prompt):
    names_i = names.astype(jnp.int32)
    prompt_flat = prompt.reshape(-1)
    return _align_prompt(graph_embedding, names_i, prompt_flat)


# speedup vs baseline: 2.1889x; 1.0019x over previous
import functools

import jax
import jax.numpy as jnp
from jax import lax
from jax.experimental import pallas as pl
from jax.experimental.pallas import tpu as pltpu
from jax.experimental.pallas import tpu_sc as plsc

N = 100000
D = 128
NDOM = 16
NC = 2
NS = 16
L = 16
NW = NC * NS
C = 160
NCHUNK = N // C
NBUF = 3
SLOTS = 21
ROUNDS = SLOTS // NBUF
PF = 2
NCG = D // L

_mesh = plsc.VectorSubcoreMesh(core_axis_name="c", subcore_axis_name="s")


@functools.partial(
    pl.kernel,
    out_type=jax.ShapeDtypeStruct((N, D), jnp.float32),
    mesh=_mesh,
    compiler_params=pltpu.CompilerParams(needs_layout_passes=False),
    scratch_types=(
        [pltpu.VMEM((NBUF, C, D), jnp.float32)]
        + [pltpu.VMEM((NBUF, C, D), jnp.float32)]
        + [pltpu.VMEM((C,), jnp.int32)] * NBUF
        + [pltpu.VMEM((NDOM * D,), jnp.float32)]
        + [pltpu.SemaphoreType.DMA] * NBUF
        + [pltpu.SemaphoreType.DMA] * NBUF
    ),
)
def _align_prompt(emb_hbm, names_hbm, prompt_hbm, out_hbm,
                  emb_v, out_v, *rest):
    names_v = rest[:NBUF]
    prompt_v = rest[NBUF]
    in_sems = rest[NBUF + 1:NBUF + 1 + NBUF]
    out_sems = rest[NBUF + 1 + NBUF:]
    wid = lax.axis_index("s") * NC + lax.axis_index("c")

    iota = lax.iota(jnp.int32, L)
    cols = [iota + (c * L) for c in range(NCG)]

    def in_copy(b, k):
        base = k * C
        return (
            pltpu.make_async_copy(names_hbm.at[pl.ds(base, C)], names_v[b],
                                  in_sems[b]),
            pltpu.make_async_copy(emb_hbm.at[pl.ds(base, C), :], emb_v.at[b],
                                  in_sems[b]),
        )

    def out_copy(b, k):
        base = k * C
        return pltpu.make_async_copy(out_v.at[b],
                                     out_hbm.at[pl.ds(base, C), :],
                                     out_sems[b])

    def compute(b):
        nref = names_v[b]

        @plsc.parallel_loop(0, C, unroll=1)
        def _row(r):
            n16 = plsc.load_gather(nref, [jnp.full((L,), r, jnp.int32)])
            bvec = n16 * D
            for c in range(NCG):
                sel = plsc.load_gather(prompt_v, [bvec + cols[c]])
                x = emb_v[b, r, pl.ds(c * L, L)]
                out_v[b, r, pl.ds(c * L, L)] = x * sel

    for s in range(PF):
        for cp in in_copy(s, wid + NW * s):
            cp.start()
    pltpu.sync_copy(prompt_hbm, prompt_v)

    def ring_round(j, _):
        for b in range(NBUF):
            s = NBUF * j + b
            k = wid + NW * s

            @pl.when(k + PF * NW < NCHUNK)
            def _(b=b, k=k):
                for cp in in_copy((b + PF) % NBUF, k + PF * NW):
                    cp.start()

            @pl.when(s >= NBUF)
            def _(b=b, k=k):
                out_copy(b, k - NBUF * NW).wait()

            @pl.when(k < NCHUNK)
            def _(b=b, k=k):
                for cp in in_copy(b, k):
                    cp.wait()
                compute(b)
                out_copy(b, k).start()
        return 0

    lax.fori_loop(0, ROUNDS, ring_round, 0, unroll=False)

    for s in range(SLOTS - NBUF, SLOTS):
        k = wid + NW * s

        @pl.when(k < NCHUNK)
        def _(s=s, k=k):
            out_copy(s % NBUF, k).wait()


def kernel(graph_embedding, names, prompt):
    names_i = names.astype(jnp.int32)
    prompt_flat = prompt.reshape(-1)
    return _align_prompt(graph_embedding, names_i, prompt_flat)
